# baseline (speedup 1.0000x reference)
import jax
import jax.numpy as jnp
from jax import lax
from jax.experimental import pallas as pl
from jax.experimental.pallas import tpu as pltpu

_B = 4
_N = 20000
_K = 1024
_DH = 256
_PAD = 20224
_H = _PAD // 2


def _mlp1_body(f_ref, w1_ref, b1_ref, h_ref):
    f = f_ref[...]
    w = w1_ref[...]
    h = (f[:, 0:1] * w[0:1, :] + f[:, 1:2] * w[1:2, :]
         + f[:, 2:3] * w[2:3, :]) + b1_ref[...]
    h_ref[...] = jnp.maximum(h, 0.0)


def _mlp2_body(h_ref, p_ref, w2_ref, b2_ref, w3_ref, b3_ref, o_ref):
    a = h_ref[...] + p_ref[...]
    a = jnp.dot(a, w2_ref[...], preferred_element_type=jnp.float32) + b2_ref[...]
    a = jnp.maximum(a, 0.0)
    o_ref[...] = jnp.dot(a, w3_ref[...], preferred_element_type=jnp.float32) + b3_ref[...]


def _pair(x, parity, combine):
    up = jnp.concatenate([x[1:], x[:1]], axis=0)
    dn = jnp.concatenate([x[-1:], x[:-1]], axis=0)
    partner = jnp.where(parity == 0, up, dn)
    return combine(x, partner)


def _fps_body(px_ref, py_ref, pz_ref, out_ref, dist_ref):
    lanes = lax.broadcasted_iota(jnp.int32, (2 * _B, _H), 1)
    parity = lax.broadcasted_iota(jnp.int32, (2 * _B, 1), 0) % 2
    gidx = lanes + parity * _H
    valid = gidx < _N
    dist_ref[...] = jnp.where(valid, jnp.float32(1e10), jnp.float32(-1e30))
    out_ref[...] = jnp.zeros((2 * _B, _K), jnp.int32)
    cols = lax.broadcasted_iota(jnp.int32, (2 * _B, _K), 1)
    ninf = jnp.float32(-jnp.inf)

    def body(i, carry):
        xl, yl, zl = carry
        dx = px_ref[...] - xl
        dy = py_ref[...] - yl
        dz = pz_ref[...] - zl
        d = dx * dx + (dy * dy + dz * dz)
        dist = jnp.minimum(dist_ref[...], d)
        dist_ref[...] = dist
        m = _pair(jnp.max(dist, axis=1, keepdims=True), parity, jnp.maximum)
        idx = _pair(jnp.min(jnp.where(dist == m, gidx, 2 ** 30), axis=1,
                            keepdims=True), parity, jnp.minimum)
        sel = gidx == idx
        out_ref[...] = jnp.where(cols == i, idx, out_ref[...])
        xl = _pair(jnp.max(jnp.where(sel, px_ref[...], ninf), axis=1,
                           keepdims=True), parity, jnp.maximum)
        yl = _pair(jnp.max(jnp.where(sel, py_ref[...], ninf), axis=1,
                           keepdims=True), parity, jnp.maximum)
        zl = _pair(jnp.max(jnp.where(sel, pz_ref[...], ninf), axis=1,
                           keepdims=True), parity, jnp.maximum)
        return (xl, yl, zl)

    def bcast0(r):
        c = r[:, 0:1]
        return jnp.where(parity == 0, c, jnp.concatenate([c[-1:], c[:-1]], axis=0))

    x0 = bcast0(px_ref)
    y0 = bcast0(py_ref)
    z0 = bcast0(pz_ref)
    lax.fori_loop(1, _K, body, (x0, y0, z0))


def kernel(points, coords, feats, inds, W1, b1, W2, b2, W3, b3):
    R = 2000
    G = (_B * _N) // R
    h = pl.pallas_call(
        _mlp1_body,
        grid=(G,),
        in_specs=[
            pl.BlockSpec((R, 3), lambda i: (i, 0)),
            pl.BlockSpec((3, _DH), lambda i: (0, 0)),
            pl.BlockSpec((1, _DH), lambda i: (0, 0)),
        ],
        out_specs=pl.BlockSpec((R, _DH), lambda i: (i, 0)),
        out_shape=jax.ShapeDtypeStruct((_B * _N, _DH), jnp.float32),
    )(feats, W1, b1.reshape(1, _DH))

    batch_ids = coords[:, 0]
    coarse = coords[:, 1:] // 2
    keys = ((batch_ids * 64 + coarse[:, 0]) * 64 + coarse[:, 1]) * 64 + coarse[:, 2]
    _, inv = jnp.unique(keys, return_inverse=True, size=keys.shape[0], fill_value=0)
    inv = inv.reshape(-1)
    S = _B * _N
    sums = jax.ops.segment_sum(h, inv, num_segments=S)
    cnts = jax.ops.segment_sum(jnp.ones((S, 1), jnp.float32), inv, num_segments=S)
    pooled = (sums / jnp.maximum(cnts, 1.0))[inv]

    features = pl.pallas_call(
        _mlp2_body,
        grid=(G,),
        in_specs=[
            pl.BlockSpec((R, _DH), lambda i: (i, 0)),
            pl.BlockSpec((R, _DH), lambda i: (i, 0)),
            pl.BlockSpec((_DH, _DH), lambda i: (0, 0)),
            pl.BlockSpec((1, _DH), lambda i: (0, 0)),
            pl.BlockSpec((_DH, _DH), lambda i: (0, 0)),
            pl.BlockSpec((1, _DH), lambda i: (0, 0)),
        ],
        out_specs=pl.BlockSpec((R, _DH), lambda i: (i, 0)),
        out_shape=jax.ShapeDtypeStruct((_B * _N, _DH), jnp.float32),
    )(h, pooled, W2, b2.reshape(1, _DH), W3, b3.reshape(1, _DH))

    inds2 = inds.reshape(_B, _N)
    pv = jnp.take_along_axis(points, inds2[..., None], axis=1)
    pvp = jnp.pad(pv, ((0, 0), (0, _PAD - _N), (0, 0)))
    px = pvp[..., 0].reshape(2 * _B, _H)
    py = pvp[..., 1].reshape(2 * _B, _H)
    pz = pvp[..., 2].reshape(2 * _B, _H)

    samp8 = pl.pallas_call(
        _fps_body,
        out_shape=jax.ShapeDtypeStruct((2 * _B, _K), jnp.int32),
        scratch_shapes=[pltpu.VMEM((2 * _B, _H), jnp.float32)],
    )(px, py, pz)
    samp = samp8[::2]

    fp2_inds = jnp.take_along_axis(inds2, samp, axis=1)
    fp2_xyz = jnp.take_along_axis(pv, samp[..., None], axis=1)
    feat4 = features.reshape(_B, _N, _DH)
    fp2_features = jnp.take_along_axis(feat4, samp[..., None], axis=1).transpose(0, 2, 1)
    return fp2_features, fp2_xyz, fp2_inds
